# Initial kernel scaffold; baseline (speedup 1.0000x reference)
import jax
import jax.numpy as jnp
from jax.experimental import pallas as pl


def kernel(x, edge_index, edge_attr, batch, W_node, b_node, W_edge0, b_edge0, W1_0, b1_0, W2_0, b2_0, W_edge1, b_edge1, W1_1, b1_1, W2_1, b2_1, W_edge2, b_edge2, W1_2, b1_2, W2_2, b2_2, W_read, b_read, Wp1, bp1, a1, Wp2, bp2, a2, Wp3, bp3):
    raise NotImplementedError("write your pallas kernel here")


# speedup vs baseline: 2.4506x; 2.4506x over previous
import functools

import jax
import jax.numpy as jnp
from jax import lax
from jax.experimental import pallas as pl
from jax.experimental.pallas import tpu as pltpu
from jax.experimental.pallas import tpu_sc as plsc

N = 10000
E = 320000
B = 64
NF = 128
EF = 16
H = 300
EMB = 1024
PH = 512

NP = 10240
HP = 320
HH = HP // 2
H2P = 640
NC = 2
NS = 16
CK = 40
CPT = 500
EPT = CPT * CK
EP = NS * EPT
ROWS_PER_TILE = NP // NS
F32 = jnp.float32


def _relu(v):
    return jnp.maximum(v, 0.0)


def _h0_body(x_ref, w_ref, b_ref, o_ref):
    o_ref[...] = jnp.dot(x_ref[...], w_ref[0],
                         preferred_element_type=F32) + b_ref[0]


def _h0_call(xp, wn3, bn3):
    return pl.pallas_call(
        _h0_body,
        grid=(NC, NP // 640),
        in_specs=[
            pl.BlockSpec((640, NF), lambda c, i: (i, 0)),
            pl.BlockSpec((1, NF, HH), lambda c, i: (c, 0, 0)),
            pl.BlockSpec((1, 1, HH), lambda c, i: (c, 0, 0)),
        ],
        out_specs=pl.BlockSpec((640, HH), lambda c, i: (c * (NP // 640) + i, 0)),
        out_shape=jax.ShapeDtypeStruct((NC * NP, HH), F32),
    )(xp, wn3, bn3)


def _edge_body(ea_ref, w0_ref, b0_ref, w1_ref, b1_ref, w2_ref, b2_ref,
               o0_ref, o1_ref, o2_ref):
    ea = ea_ref[...]
    o0_ref[...] = _relu(jnp.dot(ea, w0_ref[0], preferred_element_type=F32)
                        + b0_ref[0])
    o1_ref[...] = _relu(jnp.dot(ea, w1_ref[0], preferred_element_type=F32)
                        + b1_ref[0])
    o2_ref[...] = _relu(jnp.dot(ea, w2_ref[0], preferred_element_type=F32)
                        + b2_ref[0])


def _edge_call(eap, we, be):
    eblk = 2000
    nblk = EP // eblk
    wspec = pl.BlockSpec((1, EF, HH), lambda c, j: (c, 0, 0))
    bspec = pl.BlockSpec((1, 1, HH), lambda c, j: (c, 0, 0))
    ospec = pl.BlockSpec((eblk, HH), lambda c, j: (c * nblk + j, 0))
    oshape = jax.ShapeDtypeStruct((NC * EP, HH), F32)
    return pl.pallas_call(
        _edge_body,
        grid=(NC, nblk),
        in_specs=[pl.BlockSpec((eblk, EF), lambda c, j: (j, 0)),
                  wspec, bspec, wspec, bspec, wspec, bspec],
        out_specs=[ospec, ospec, ospec],
        out_shape=[oshape, oshape, oshape],
    )(eap, we[0], be[0], we[1], be[1], we[2], be[2])


def _node_body(aggA_ref, aggB_ref, hA_ref, hB_ref, w1a_ref, w1b_ref, b1_ref,
               w2_ref, b2_ref, o_ref):
    zA = aggA_ref[...] + hA_ref[...]
    zB = aggB_ref[...] + hB_ref[...]
    t = _relu(jnp.dot(zA, w1a_ref[...], preferred_element_type=F32)
              + jnp.dot(zB, w1b_ref[...], preferred_element_type=F32)
              + b1_ref[...])
    o_ref[...] = _relu(jnp.dot(t, w2_ref[0], preferred_element_type=F32)
                       + b2_ref[0])


def _node_call(agg, h, w1a, w1b, b1p, w23, b23):
    nb = NP // 640
    halfA = pl.BlockSpec((640, HH), lambda c, i: (i, 0))
    halfB = pl.BlockSpec((640, HH), lambda c, i: (nb + i, 0))
    return pl.pallas_call(
        _node_body,
        grid=(NC, nb),
        in_specs=[
            halfA, halfB, halfA, halfB,
            pl.BlockSpec((HH, H2P), lambda c, i: (0, 0)),
            pl.BlockSpec((HH, H2P), lambda c, i: (0, 0)),
            pl.BlockSpec((1, H2P), lambda c, i: (0, 0)),
            pl.BlockSpec((1, H2P, HH), lambda c, i: (c, 0, 0)),
            pl.BlockSpec((1, 1, HH), lambda c, i: (c, 0, 0)),
        ],
        out_specs=pl.BlockSpec((640, HH), lambda c, i: (c * nb + i, 0)),
        out_shape=jax.ShapeDtypeStruct((NC * NP, HH), F32),
    )(agg, agg, h, h, w1a, w1b, b1p, w23, b23)


def _batch_body(h_ref, bt_ref, o_ref):
    i = pl.program_id(1)

    @pl.when(i == 0)
    def _():
        o_ref[...] = jnp.zeros_like(o_ref)

    rows = h_ref.shape[0]
    seg = jnp.broadcast_to(bt_ref[...], (B, rows))
    ids = lax.broadcasted_iota(jnp.int32, (B, rows), 0)
    onehot = (seg == ids).astype(F32)
    o_ref[...] += jnp.dot(onehot, h_ref[...], preferred_element_type=F32)


def _batch_call(h, batch2d):
    nb = NP // 640
    return pl.pallas_call(
        _batch_body,
        grid=(NC, nb),
        in_specs=[
            pl.BlockSpec((640, HH), lambda c, i: (c * nb + i, 0)),
            pl.BlockSpec((1, 640), lambda c, i: (0, i)),
        ],
        out_specs=pl.BlockSpec((B, HH), lambda c, i: (c, 0)),
        out_shape=jax.ShapeDtypeStruct((NC * B, HH), F32),
    )(h, batch2d)


def _read_body(gA_ref, gB_ref, wra_ref, wrb_ref, br_ref, wp1_ref, bp1_ref,
               a1_ref, wp2_ref, bp2_ref, a2_ref, wp3_ref, bp3_ref, o_ref):
    g = _relu(jnp.dot(gA_ref[...], wra_ref[...], preferred_element_type=F32)
              + jnp.dot(gB_ref[...], wrb_ref[...], preferred_element_type=F32)
              + br_ref[...])
    o = jnp.dot(g, wp1_ref[...], preferred_element_type=F32) + bp1_ref[...]
    o = jnp.where(o > 0, o, a1_ref[0, 0] * o)
    o = jnp.dot(o, wp2_ref[...], preferred_element_type=F32) + bp2_ref[...]
    o = jnp.where(o > 0, o, a2_ref[0, 0] * o)
    o_ref[...] = jnp.dot(o, wp3_ref[...], preferred_element_type=F32) + bp3_ref[...]


def _read_call(g, wra, wrb, br, wp1, bp1, a1, wp2, bp2, a2, wp3p, bp3p):
    full = lambda *shape: pl.BlockSpec(shape, lambda i: tuple(0 for _ in shape))
    return pl.pallas_call(
        _read_body,
        grid=(1,),
        in_specs=[
            pl.BlockSpec((B, HH), lambda i: (0, 0)),
            pl.BlockSpec((B, HH), lambda i: (1, 0)),
            full(HH, EMB), full(HH, EMB), full(1, EMB),
            full(EMB, PH), full(1, PH), full(1, 1),
            full(PH, PH), full(1, PH), full(1, 1),
            full(PH, 128), full(1, 128),
        ],
        out_specs=full(B, 128),
        out_shape=jax.ShapeDtypeStruct((B, 128), F32),
    )(g, g, wra, wrb, br, wp1, bp1, a1, wp2, bp2, a2, wp3p, bp3p)


def _sc_body(h_hbm, e_hbm, src_hbm, dst_hbm, out_hbm,
             sidx, didx, hbuf, ebuf, agg,
             gsem0, gsem1, esem0, esem1, isem0, isem1, isem2, isem3):
    c = lax.axis_index("c")
    s = lax.axis_index("s")

    def _zrow(r, carry):
        for f in range(HH // 16):
            ebuf[0, r, pl.ds(f * 16, 16)] = jnp.zeros((16,), F32)
        return carry
    lax.fori_loop(0, CK, _zrow, 0)
    for q in range(ROWS_PER_TILE // CK):
        pltpu.sync_copy(ebuf.at[0],
                        agg.at[pl.ds(s * ROWS_PER_TILE + q * CK, CK)])
    plsc.subcore_barrier()

    ebase = c * EP + s * EPT
    ibase = (c * NS + s) * CPT
    dbase = s * CPT
    gsems = (gsem0, gsem1)
    esems = (esem0, esem1)
    isems = (isem0, isem1, isem2, isem3)

    def _stage_idx(j, slot):
        pltpu.async_copy(src_hbm.at[ibase + j], sidx.at[slot], isems[slot])
        pltpu.async_copy(dst_hbm.at[dbase + j], didx.at[slot], isems[slot])

    def _wait_idx(j, slot):
        pltpu.make_async_copy(src_hbm.at[ibase + j], sidx.at[slot],
                              isems[slot]).wait()
        pltpu.make_async_copy(dst_hbm.at[dbase + j], didx.at[slot],
                              isems[slot]).wait()

    def _issue(j, b, slot):
        pltpu.async_copy(h_hbm.at[sidx.at[slot]], hbuf.at[b], gsems[b])
        pltpu.async_copy(e_hbm.at[pl.ds(ebase + j * CK, CK)], ebuf.at[b],
                         esems[b])

    def _wait(j, b, slot):
        pltpu.make_async_copy(h_hbm.at[sidx.at[slot]], hbuf.at[b],
                              gsems[b]).wait()
        pltpu.make_async_copy(e_hbm.at[pl.ds(ebase + j * CK, CK)],
                              ebuf.at[b], esems[b]).wait()

    for j in range(4):
        _stage_idx(j, j)
    _wait_idx(0, 0)
    _wait_idx(1, 1)
    _issue(0, 0, 0)
    _issue(1, 1, 1)

    def _quad(j4, carry):
        for u in range(4):
            j = j4 * 4 + u
            b = u % 2
            _wait(j, b, u)

            def _crow(r, cc):
                for f in range(HH // 16):
                    sl = pl.ds(f * 16, 16)
                    hbuf[b, r, sl] = jnp.maximum(
                        hbuf[b, r, sl] + ebuf[b, r, sl], 0.0)
                return cc
            lax.fori_loop(0, CK, _crow, 0)

            pltpu.sync_copy(hbuf.at[b], agg.at[didx.at[u]], add=True)

            @pl.when(j + 4 < CPT)
            def _():
                _stage_idx(j + 4, u)

            @pl.when(j + 2 < CPT)
            def _():
                nslot = (u + 2) % 4
                _wait_idx(j + 2, nslot)
                _issue(j + 2, b, nslot)
        return carry

    lax.fori_loop(0, CPT // 4, _quad, 0)
    plsc.subcore_barrier()

    for q in range(ROWS_PER_TILE // CK):
        base = s * ROWS_PER_TILE + q * CK
        pltpu.sync_copy(agg.at[pl.ds(base, CK)], ebuf.at[0])
        pltpu.sync_copy(ebuf.at[0], out_hbm.at[pl.ds(c * NP + base, CK)])


def _sc_call(h, e, src2, dst3):
    mesh = plsc.VectorSubcoreMesh(core_axis_name="c", subcore_axis_name="s",
                                  num_cores=NC, num_subcores=NS)
    fn = pl.kernel(
        _sc_body,
        out_type=jax.ShapeDtypeStruct((NC * NP, HH), F32),
        mesh=mesh,
        scratch_types=[
            pltpu.VMEM((4, CK), jnp.int32),
            pltpu.VMEM((4, CK), jnp.int32),
            pltpu.VMEM((2, CK, HH), F32),
            pltpu.VMEM((2, CK, HH), F32),
            pltpu.VMEM_SHARED((NP, HH), F32),
            pltpu.SemaphoreType.DMA,
            pltpu.SemaphoreType.DMA,
            pltpu.SemaphoreType.DMA,
            pltpu.SemaphoreType.DMA,
            pltpu.SemaphoreType.DMA,
            pltpu.SemaphoreType.DMA,
            pltpu.SemaphoreType.DMA,
            pltpu.SemaphoreType.DMA,
        ],
        compiler_params=pltpu.CompilerParams(use_tc_tiling_on_sc=False),
    )
    return fn(h, e, src2, dst3)


def _split_cols(wmat, bvec):
    w3 = jnp.stack([wmat[:, :HH], wmat[:, HH:]])
    b3 = jnp.stack([bvec[:HH], bvec[HH:]]).reshape(2, 1, HH)
    return w3, b3


def kernel(x, edge_index, edge_attr, batch, W_node, b_node, W_edge0, b_edge0,
           W1_0, b1_0, W2_0, b2_0, W_edge1, b_edge1, W1_1, b1_1, W2_1, b2_1,
           W_edge2, b_edge2, W1_2, b1_2, W2_2, b2_2, W_read, b_read, Wp1, bp1,
           a1, Wp2, bp2, a2, Wp3, bp3):
    xp = jnp.pad(x, ((0, NP - N), (0, 0)))
    src = edge_index[0]
    dst = edge_index[1]
    src2 = jnp.stack([src, src + NP]).reshape(NC * NS * CPT, CK)
    dst3 = dst.reshape(NS * CPT, CK)
    eap = edge_attr
    batch2d = jnp.pad(batch, (0, NP - N), constant_values=B).reshape(1, NP)

    wn3, bn3 = _split_cols(jnp.pad(W_node, ((0, 0), (0, HP - H))),
                           jnp.pad(b_node, (0, HP - H)))
    we = []
    beb = []
    for W_e, b_e in ((W_edge0, b_edge0), (W_edge1, b_edge1),
                     (W_edge2, b_edge2)):
        w3, b3 = _split_cols(jnp.pad(W_e, ((0, 0), (0, HP - H))),
                             jnp.pad(b_e, (0, HP - H)))
        we.append(w3)
        beb.append(b3)

    layer_mlp = []
    for W1, b1, W2, b2 in ((W1_0, b1_0, W2_0, b2_0),
                           (W1_1, b1_1, W2_1, b2_1),
                           (W1_2, b1_2, W2_2, b2_2)):
        w1p = jnp.pad(W1, ((0, HP - H), (0, H2P - 2 * H)))
        b1p = jnp.pad(b1, (0, H2P - 2 * H)).reshape(1, H2P)
        w23, b23 = _split_cols(jnp.pad(W2, ((0, H2P - 2 * H), (0, HP - H))),
                               jnp.pad(b2, (0, HP - H)))
        layer_mlp.append((w1p[:HH], w1p[HH:], b1p, w23, b23))

    wrp = jnp.pad(W_read, ((0, HP - H), (0, 0)))
    wra, wrb = wrp[:HH], wrp[HH:]
    br = b_read.reshape(1, EMB)
    wp3p = jnp.pad(Wp3, ((0, 0), (0, 127)))
    bp3p = jnp.pad(bp3, (0, 127)).reshape(1, 128)

    h = _h0_call(xp, wn3, bn3)
    e0, e1, e2 = _edge_call(eap, we, beb)
    for el, (w1a, w1b, b1p, w23, b23) in zip((e0, e1, e2), layer_mlp):
        agg = _sc_call(h, el, src2, dst3)
        h = _node_call(agg, h, w1a, w1b, b1p, w23, b23)
    g = _batch_call(h, batch2d)
    out = _read_call(g, wra, wrb, br, Wp1, bp1.reshape(1, PH),
                     a1.reshape(1, 1), Wp2, bp2.reshape(1, PH),
                     a2.reshape(1, 1), wp3p, bp3p)
    return out[:, :1]
